# baseline (speedup 1.0000x reference)
import jax
import jax.numpy as jnp
from jax.experimental import pallas as pl
from jax.experimental.pallas import tpu as pltpu

_N = 6
_C = 64
_TB = 1024
_MISS = {2: 5, 3: 4, 4: 3, 5: 2}


def _erf_gelu_scaled(m):
    return m * (1.0 + jax.lax.erf(m))


def _layer(x2d, wr_ref, d_ref, b_ref):
    w_all = jnp.dot(x2d, wr_ref[...], preferred_element_type=jnp.float32)
    u_all = jnp.dot(x2d, d_ref[...], preferred_element_type=jnp.float32)
    w = [w_all[i * _TB:(i + 1) * _TB] for i in range(_N)]
    u = [u_all[i * _TB:(i + 1) * _TB] for i in range(_N)]
    t = w[0] + w[1] + w[2] + w[3] + w[4] + w[5] + b_ref[...]
    out = []
    for i in range(_N):
        m = t + u[i]
        if i in _MISS:
            m = m - w[_MISS[i]]
        out.append(_erf_gelu_scaled(m))
    return jnp.concatenate(out, axis=0)


def _body(x_ref, wr1_ref, d1_ref, b1_ref, wr2_ref, d2_ref, b2_ref,
          gs_ref, bt_ref, o_ref):
    x2d = x_ref[...].reshape(_N * _TB, _C)
    g1 = _layer(x2d, wr1_ref, d1_ref, b1_ref)
    g2 = _layer(g1, wr2_ref, d2_ref, b2_ref)
    gs = [g2[i * _TB:(i + 1) * _TB] for i in range(_N)]
    acc = gs[0] + gs[1] + gs[2] + gs[3] + gs[4] + gs[5]
    o_ref[...] = acc * gs_ref[...] + bt_ref[...]


@jax.jit
def kernel(x, W_rel1, b_rel1, W_root1, W_rel2, b_rel2, W_root2, gamma, beta):
    n, b, c = x.shape
    rc = 0.7071067811865476
    wr1 = rc * W_rel1.T
    d1 = rc * (W_root1.T - W_rel1.T)
    b1 = (rc * b_rel1).reshape(1, c)
    wr2 = 0.5 * W_rel2.T
    d2 = 0.5 * (W_root2.T - W_rel2.T)
    b2 = (rc * b_rel2).reshape(1, c)
    gs = (gamma * (rc / (n * jnp.sqrt(1.0 + 1e-5)))).reshape(1, c)
    bt = beta.reshape(1, c)

    grid = (b // _TB,)
    wspec = pl.BlockSpec((c, c), lambda i: (0, 0))
    vspec = pl.BlockSpec((1, c), lambda i: (0, 0))
    return pl.pallas_call(
        _body,
        grid=grid,
        in_specs=[
            pl.BlockSpec((n, _TB, c), lambda i: (0, i, 0)),
            wspec, wspec, vspec,
            wspec, wspec, vspec,
            vspec, vspec,
        ],
        out_specs=pl.BlockSpec((_TB, c), lambda i: (i, 0)),
        out_shape=jax.ShapeDtypeStruct((b, c), jnp.float32),
        compiler_params=pltpu.CompilerParams(
            dimension_semantics=("parallel",),
        ),
    )(x, wr1, d1, b1, wr2, d2, b2, gs, bt)


# speedup vs baseline: 1.0560x; 1.0560x over previous
import jax
import jax.numpy as jnp
from jax.experimental import pallas as pl
from jax.experimental.pallas import tpu as pltpu

_N = 6
_C = 64
_TB = 1024
_MISS = {2: 5, 3: 4, 4: 3, 5: 2}


def _erf_gelu_scaled(m):
    return m * (1.0 + jax.lax.erf(m))


def _layer(xs, wr_ref, d_ref, b_ref):
    wr = wr_ref[...]
    d = d_ref[...]
    s = xs[0] + xs[1] + xs[2] + xs[3] + xs[4] + xs[5]
    t = jnp.dot(s, wr, preferred_element_type=jnp.float32) + b_ref[...]
    u = [jnp.dot(xi, d, preferred_element_type=jnp.float32) for xi in xs]
    v = {j: jnp.dot(xs[j], wr, preferred_element_type=jnp.float32)
         for j in _MISS}
    out = []
    for i in range(_N):
        m = t + u[i]
        if i in _MISS:
            m = m - v[_MISS[i]]
        out.append(_erf_gelu_scaled(m))
    return out


def _body(x_ref, wr1_ref, d1_ref, b1_ref, wr2_ref, d2_ref, b2_ref,
          gs_ref, bt_ref, o_ref):
    xs = [x_ref[i] for i in range(_N)]
    g1 = _layer(xs, wr1_ref, d1_ref, b1_ref)
    g2 = _layer(g1, wr2_ref, d2_ref, b2_ref)
    acc = g2[0] + g2[1] + g2[2] + g2[3] + g2[4] + g2[5]
    o_ref[...] = acc * gs_ref[...] + bt_ref[...]


@jax.jit
def kernel(x, W_rel1, b_rel1, W_root1, W_rel2, b_rel2, W_root2, gamma, beta):
    n, b, c = x.shape
    rc = 0.7071067811865476
    wr1 = rc * W_rel1.T
    d1 = rc * (W_root1.T - W_rel1.T)
    b1 = (rc * b_rel1).reshape(1, c)
    wr2 = 0.5 * W_rel2.T
    d2 = 0.5 * (W_root2.T - W_rel2.T)
    b2 = (rc * b_rel2).reshape(1, c)
    gs = (gamma * (rc / (n * jnp.sqrt(1.0 + 1e-5)))).reshape(1, c)
    bt = beta.reshape(1, c)

    grid = (b // _TB,)
    wspec = pl.BlockSpec((c, c), lambda i: (0, 0))
    vspec = pl.BlockSpec((1, c), lambda i: (0, 0))
    return pl.pallas_call(
        _body,
        grid=grid,
        in_specs=[
            pl.BlockSpec((n, _TB, c), lambda i: (0, i, 0)),
            wspec, wspec, vspec,
            wspec, wspec, vspec,
            vspec, vspec,
        ],
        out_specs=pl.BlockSpec((_TB, c), lambda i: (i, 0)),
        out_shape=jax.ShapeDtypeStruct((b, c), jnp.float32),
        compiler_params=pltpu.CompilerParams(
            dimension_semantics=("parallel",),
        ),
    )(x, wr1, d1, b1, wr2, d2, b2, gs, bt)


# speedup vs baseline: 1.0567x; 1.0007x over previous
import jax
import jax.numpy as jnp
from jax.experimental import pallas as pl
from jax.experimental.pallas import tpu as pltpu

_N = 6
_C = 64
_TB = 1024
_MISS = {2: 5, 3: 4, 4: 3, 5: 2}


def _erf_gelu_scaled(m):
    return m * (1.0 + jax.lax.erf(m))


def _layer(xs, wr_ref, d_ref, b_ref):
    wr = wr_ref[...]
    d = d_ref[...]
    s = xs[0] + xs[1] + xs[2] + xs[3] + xs[4] + xs[5]
    xb = [xi.astype(jnp.bfloat16) for xi in xs]
    sb = s.astype(jnp.bfloat16)
    t = jnp.dot(sb, wr, preferred_element_type=jnp.float32) + b_ref[...]
    u = [jnp.dot(xi, d, preferred_element_type=jnp.float32) for xi in xb]
    v = {j: jnp.dot(xb[j], wr, preferred_element_type=jnp.float32)
         for j in _MISS}
    out = []
    for i in range(_N):
        m = t + u[i]
        if i in _MISS:
            m = m - v[_MISS[i]]
        out.append(_erf_gelu_scaled(m))
    return out


def _body(x_ref, wr1_ref, d1_ref, b1_ref, wr2_ref, d2_ref, b2_ref,
          gs_ref, bt_ref, o_ref):
    xs = [x_ref[i] for i in range(_N)]
    g1 = _layer(xs, wr1_ref, d1_ref, b1_ref)
    g2 = _layer(g1, wr2_ref, d2_ref, b2_ref)
    acc = g2[0] + g2[1] + g2[2] + g2[3] + g2[4] + g2[5]
    o_ref[...] = acc * gs_ref[...] + bt_ref[...]


@jax.jit
def kernel(x, W_rel1, b_rel1, W_root1, W_rel2, b_rel2, W_root2, gamma, beta):
    n, b, c = x.shape
    rc = 0.7071067811865476
    wr1 = (rc * W_rel1.T).astype(jnp.bfloat16)
    d1 = (rc * (W_root1.T - W_rel1.T)).astype(jnp.bfloat16)
    b1 = (rc * b_rel1).reshape(1, c)
    wr2 = (0.5 * W_rel2.T).astype(jnp.bfloat16)
    d2 = (0.5 * (W_root2.T - W_rel2.T)).astype(jnp.bfloat16)
    b2 = (rc * b_rel2).reshape(1, c)
    gs = (gamma * (rc / (n * jnp.sqrt(1.0 + 1e-5)))).reshape(1, c)
    bt = beta.reshape(1, c)

    grid = (b // _TB,)
    wspec = pl.BlockSpec((c, c), lambda i: (0, 0))
    vspec = pl.BlockSpec((1, c), lambda i: (0, 0))
    return pl.pallas_call(
        _body,
        grid=grid,
        in_specs=[
            pl.BlockSpec((n, _TB, c), lambda i: (0, i, 0)),
            wspec, wspec, vspec,
            wspec, wspec, vspec,
            vspec, vspec,
        ],
        out_specs=pl.BlockSpec((_TB, c), lambda i: (i, 0)),
        out_shape=jax.ShapeDtypeStruct((b, c), jnp.float32),
        compiler_params=pltpu.CompilerParams(
            dimension_semantics=("parallel",),
        ),
    )(x, wr1, d1, b1, wr2, d2, b2, gs, bt)
